# baseline (speedup 1.0000x reference)
import functools

import jax
import jax.numpy as jnp
from jax import lax
from jax.experimental import pallas as pl
from jax.experimental.pallas import tpu as pltpu
from jax.experimental.pallas import tpu_sc as plsc

B = 16384
L = 50
D = 64
NBT = B // 128

_LOG1P_COEF = (
    9.97503255195653e-06,
    0.9992354838332754,
    -0.4902307234234105,
    0.285272681090574,
    -0.13158182508876004,
    0.03044900453866886,
)


def _softplus16(v):
    t = jnp.exp(-jnp.abs(v))
    p = jnp.float32(_LOG1P_COEF[-1])
    for c in reversed(_LOG1P_COEF[:-1]):
        p = p * t + jnp.float32(c)
    return jnp.maximum(v, jnp.float32(0.0)) + p


@functools.lru_cache(maxsize=None)
def _make():
    info = plsc.get_sparse_core_info()
    nc, ns = info.num_cores, info.num_subcores
    nw = nc * ns
    assert nw == 32
    LW = L // 2
    BW = NBT // 16
    n_units = LW * (BW // 2)
    mesh = plsc.VectorSubcoreMesh(core_axis_name="c", subcore_axis_name="s")

    @functools.partial(
        pl.kernel,
        mesh=mesh,
        out_type=jax.ShapeDtypeStruct((L, 8, 128, 8, 128), jnp.float32),
        scratch_types=[
            pltpu.VMEM((LW * BW * 128,), jnp.int32),
            pltpu.VMEM((2, 256, D), jnp.float32),
            pltpu.VMEM((2, D, 259), jnp.float32),
            pltpu.SemaphoreType.DMA,
            pltpu.SemaphoreType.DMA,
            pltpu.SemaphoreType.DMA,
            pltpu.SemaphoreType.DMA,
            pltpu.SemaphoreType.DMA,
        ],
        compiler_params=pltpu.CompilerParams(
            use_tc_tiling_on_sc=False, needs_layout_passes=False
        ),
    )
    def k(wordT_hbm, table_hbm, out_hbm, idx_v, gbuf, obuf, isem, gs0, gs1, os0, os1):
        gsems = (gs0, gs1)
        osems = (os0, os1)
        wid = lax.axis_index("s") * nc + lax.axis_index("c")
        l0 = (wid % 2) * LW
        bt0 = (wid // 2) * BW

        for li in range(LW):
            pltpu.async_copy(
                wordT_hbm.at[l0 + li, pl.ds(bt0 * 128, BW * 128)],
                idx_v.at[pl.ds(li * BW * 128, BW * 128)],
                isem,
            )
        for li in range(LW):
            pltpu.make_async_copy(
                wordT_hbm.at[0, pl.ds(0, BW * 128)],
                idx_v.at[pl.ds(0, BW * 128)],
                isem,
            ).wait()

        def fire_gather(t, b):
            pltpu.async_copy(
                table_hbm.at[idx_v.at[pl.ds(t * 256, 256)]], gbuf.at[b], gsems[b]
            )

        def wait_gather(b):
            pltpu.make_async_copy(
                table_hbm.at[idx_v.at[pl.ds(0, 256)]], gbuf.at[b], gsems[b]
            ).wait()

        def fire_out(t, b):
            li = t // (BW // 2)
            bj = (t % (BW // 2)) * 2
            for dt in range(8):
                for jj in range(2):
                    pltpu.async_copy(
                        obuf.at[b, pl.ds(dt * 8, 8), pl.ds(jj * 128, 128)],
                        out_hbm.at[l0 + li, dt, bt0 + bj + jj],
                        osems[b],
                    )

        def wait_out(b):
            for _ in range(16):
                pltpu.make_async_copy(
                    obuf.at[b, pl.ds(0, 8), pl.ds(0, 128)],
                    out_hbm.at[0, 0, 0],
                    osems[b],
                ).wait()

        rows_j = [lax.iota(jnp.int32, 16) + j * 16 for j in range(4)]

        def compute(b):
            @plsc.parallel_loop(0, 256, unroll=4)
            def col(kk):
                colk = jnp.broadcast_to(kk, (16,))
                for j in range(4):
                    v = gbuf[b, kk, pl.ds(j * 16, 16)]
                    plsc.store_scatter(
                        obuf.at[b], [rows_j[j], colk], _softplus16(v)
                    )

        fire_gather(0, 0)
        fire_gather(1, 1)

        @pl.loop(0, n_units, step=2)
        def unit(t0):
            for bb in range(2):
                t = t0 + bb
                wait_gather(bb)

                @pl.when(t >= 2)
                def _():
                    wait_out(bb)

                compute(bb)

                @pl.when(t + 2 < n_units)
                def _():
                    fire_gather(t + 2, bb)

                fire_out(t, bb)

        wait_out(0)
        wait_out(1)

    return k


def kernel(word, emb_weight):
    wordT = word.T.astype(jnp.int32)
    out5 = _make()(wordT, emb_weight)
    return out5.transpose((2, 4, 0, 1, 3)).reshape(B, L, D)


# speedup vs baseline: 1.0160x; 1.0160x over previous
import functools

import jax
import jax.numpy as jnp
from jax import lax
from jax.experimental import pallas as pl
from jax.experimental.pallas import tpu as pltpu
from jax.experimental.pallas import tpu_sc as plsc

B = 16384
L = 50
D = 64
NBT = B // 128

_LOG1P_COEF = (
    9.97503255195653e-06,
    0.9992354838332754,
    -0.4902307234234105,
    0.285272681090574,
    -0.13158182508876004,
    0.03044900453866886,
)


def _softplus16(v):
    t = jnp.exp(-jnp.abs(v))
    p = jnp.float32(_LOG1P_COEF[-1])
    for c in reversed(_LOG1P_COEF[:-1]):
        p = p * t + jnp.float32(c)
    return jnp.maximum(v, jnp.float32(0.0)) + p


@functools.lru_cache(maxsize=None)
def _make():
    info = plsc.get_sparse_core_info()
    nc, ns = info.num_cores, info.num_subcores
    nw = nc * ns
    assert nw == 32
    LW = L // 2
    BW = NBT // 16
    n_units = LW * BW
    mesh = plsc.VectorSubcoreMesh(core_axis_name="c", subcore_axis_name="s")

    @functools.partial(
        pl.kernel,
        mesh=mesh,
        out_type=jax.ShapeDtypeStruct((L, 8, 128, 8, 128), jnp.float32),
        scratch_types=[
            pltpu.VMEM((LW * BW * 128,), jnp.int32),
            pltpu.VMEM((2, 128, D), jnp.float32),
            pltpu.VMEM((2, D, 129), jnp.float32),
            pltpu.SemaphoreType.DMA,
            pltpu.SemaphoreType.DMA,
            pltpu.SemaphoreType.DMA,
            pltpu.SemaphoreType.DMA,
            pltpu.SemaphoreType.DMA,
        ],
        compiler_params=pltpu.CompilerParams(
            use_tc_tiling_on_sc=False, needs_layout_passes=False
        ),
    )
    def k(wordT_hbm, table_hbm, out_hbm, idx_v, gbuf, obuf, isem, gs0, gs1, os0, os1):
        gsems = (gs0, gs1)
        osems = (os0, os1)
        wid = lax.axis_index("s") * nc + lax.axis_index("c")
        l0 = (wid % 2) * LW
        bt0 = (wid // 2) * BW

        for li in range(LW):
            pltpu.async_copy(
                wordT_hbm.at[l0 + li, pl.ds(bt0 * 128, BW * 128)],
                idx_v.at[pl.ds(li * BW * 128, BW * 128)],
                isem,
            )
        for li in range(LW):
            pltpu.make_async_copy(
                wordT_hbm.at[0, pl.ds(0, BW * 128)],
                idx_v.at[pl.ds(0, BW * 128)],
                isem,
            ).wait()

        def fire_gather(t, b):
            pltpu.async_copy(
                table_hbm.at[idx_v.at[pl.ds(t * 128, 128)]], gbuf.at[b], gsems[b]
            )

        def wait_gather(b):
            pltpu.make_async_copy(
                table_hbm.at[idx_v.at[pl.ds(0, 128)]], gbuf.at[b], gsems[b]
            ).wait()

        def fire_out(t, b):
            li = t // BW
            bj = t % BW
            for dt in range(8):
                pltpu.async_copy(
                    obuf.at[b, pl.ds(dt * 8, 8), pl.ds(0, 128)],
                    out_hbm.at[l0 + li, dt, bt0 + bj],
                    osems[b],
                )

        def wait_out(b):
            for dt in range(8):
                pltpu.make_async_copy(
                    obuf.at[b, pl.ds(0, 8), pl.ds(0, 128)],
                    out_hbm.at[0, 0, 0],
                    osems[b],
                ).wait()

        rows_j = [lax.iota(jnp.int32, 16) + j * 16 for j in range(4)]

        def compute(b):
            @plsc.parallel_loop(0, 128, unroll=4)
            def col(kk):
                colk = jnp.broadcast_to(kk, (16,))
                for j in range(4):
                    v = gbuf[b, kk, pl.ds(j * 16, 16)]
                    plsc.store_scatter(
                        obuf.at[b], [rows_j[j], colk], _softplus16(v)
                    )

        fire_gather(0, 0)
        fire_gather(1, 1)

        @pl.loop(0, n_units, step=2)
        def unit(t0):
            for bb in range(2):
                t = t0 + bb
                wait_gather(bb)

                @pl.when(t >= 2)
                def _():
                    wait_out(bb)

                compute(bb)

                @pl.when(t + 2 < n_units)
                def _():
                    fire_gather(t + 2, bb)

                fire_out(t, bb)

        wait_out(0)
        wait_out(1)

    return k


def kernel(word, emb_weight):
    wordT = word.T.astype(jnp.int32)
    out5 = _make()(wordT, emb_weight)
    return out5.transpose((2, 4, 0, 1, 3)).reshape(B, L, D)
